# baseline (speedup 1.0000x reference)
import functools
import math

import jax
import jax.numpy as jnp
from jax import lax
from jax.experimental import pallas as pl
from jax.experimental.pallas import tpu as pltpu
from jax.experimental.pallas import tpu_sc as plsc

N_TOTAL = 10000
M = 4
K = 8
S = N_TOTAL * M
F = S * K
E = 500000
H = 128
NH = 4
DH = H // NH
MAX_DIST = 32
EDGE_DIM = 8
NUM_GRAPHS = 64

_EPAD = 524288
_EPT = _EPAD // 16
_FPAD = 327680
_NPT = _FPAD // 16
_WORDS = _FPAD // 32
_WPT = _WORDS // 16


def _bfs_body(src_hbm, dst_hbm, out_hbm, src_c, dst_c, c_v, dist_v,
              racc_v, cnt_v, cntall_v, racc_sp, rprev_sp, cnt_sp, *,
              rounds=MAX_DIST - 1):
    cid = lax.axis_index("c")
    sid = lax.axis_index("s")
    lane = lax.iota(jnp.int32, 16)
    rootpat = jnp.where(lane % 8 == 0, 1, 0).astype(jnp.int32)
    unreach = jnp.where(lane % 8 == 0, 0, 1).astype(jnp.int32)

    def fill(i, _):
        racc_v[pl.ds(i * 16, 16)] = rootpat
        dist_v[pl.ds(i * 16, 16)] = unreach
        return 0

    lax.fori_loop(0, _NPT // 16, fill, 0)
    pltpu.sync_copy(racc_v, racc_sp.at[pl.ds(sid * _NPT, _NPT)])
    pltpu.sync_copy(racc_v, rprev_sp.at[pl.ds(sid * _NPT, _NPT)])
    plsc.subcore_barrier()

    def round_body(carry):
        t, prev_total, _ = carry
        for j in range(_EPT // 4096):
            pltpu.sync_copy(src_hbm.at[sid, j], src_c)
            pltpu.sync_copy(dst_hbm.at[sid, j], dst_c)
            pltpu.sync_copy(rprev_sp.at[src_c], c_v)
            pltpu.sync_copy(c_v, racc_sp.at[dst_c], add=True)
        plsc.subcore_barrier()
        pltpu.sync_copy(racc_sp.at[pl.ds(sid * _NPT, _NPT)], racc_v)

        def dstep(i, acc):
            b = jnp.minimum(racc_v[pl.ds(i * 16, 16)], 1)
            dist_v[pl.ds(i * 16, 16)] = dist_v[pl.ds(i * 16, 16)] + 1 - b
            racc_v[pl.ds(i * 16, 16)] = b
            return acc + b

        acc = lax.fori_loop(0, _NPT // 16, dstep, jnp.zeros((16,), jnp.int32))
        pltpu.sync_copy(racc_v, rprev_sp.at[pl.ds(sid * _NPT, _NPT)])
        cnt_v[pl.ds(0, 16)] = jnp.broadcast_to(jnp.sum(acc, axis=0), (16,))
        pltpu.sync_copy(cnt_v, cnt_sp.at[pl.ds(sid * 16, 16)])
        plsc.subcore_barrier()
        pltpu.sync_copy(cnt_sp, cntall_v)
        total = jnp.zeros((16,), jnp.int32)
        for i in range(16):
            total = total + cntall_v[pl.ds(i * 16, 16)]
        total = jnp.sum(total, axis=0)
        plsc.subcore_barrier()
        return t + 1, total, prev_total != total

    def round_cond(carry):
        t, _, changed = carry
        return jnp.logical_and(t < rounds, changed)

    t_done, _, _ = lax.while_loop(round_cond, round_body,
                                  (jnp.int32(0), jnp.int32(-1), True))
    rem = rounds - t_done

    def fixup(i, _):
        b = racc_v[pl.ds(i * 16, 16)]
        dist_v[pl.ds(i * 16, 16)] = dist_v[pl.ds(i * 16, 16)] + rem * (1 - b)
        return 0

    lax.fori_loop(0, _NPT // 16, fixup, 0)

    @pl.when(cid == 0)
    def _():
        @pl.when(sid < 15)
        def _():
            pltpu.sync_copy(dist_v, out_hbm.at[pl.ds(sid * _NPT, _NPT)])

        @pl.when(sid == 15)
        def _():
            pltpu.sync_copy(dist_v.at[pl.ds(0, F - 15 * _NPT)],
                            out_hbm.at[pl.ds(15 * _NPT, F - 15 * _NPT)])


def _bfs_dist(src, dst, rounds=MAX_DIST - 1):
    pad = _EPAD - E
    srcp = jnp.concatenate([src, jnp.zeros((pad,), src.dtype)])
    dstp = jnp.concatenate([dst, jnp.zeros((pad,), dst.dtype)])
    src3 = srcp.reshape(16, _EPT // 4096, 4096).astype(jnp.int32)
    dst3 = dstp.reshape(16, _EPT // 4096, 4096).astype(jnp.int32)
    mesh = plsc.VectorSubcoreMesh(core_axis_name="c", subcore_axis_name="s")
    f = pl.kernel(
        functools.partial(_bfs_body, rounds=rounds),
        out_type=jax.ShapeDtypeStruct((F,), jnp.int32),
        mesh=mesh,
        scratch_types=[
            pltpu.VMEM((4096,), jnp.int32),
            pltpu.VMEM((4096,), jnp.int32),
            pltpu.VMEM((4096,), jnp.int32),
            pltpu.VMEM((_NPT,), jnp.int32),
            pltpu.VMEM((_NPT,), jnp.int32),
            pltpu.VMEM((16,), jnp.int32),
            pltpu.VMEM((256,), jnp.int32),
            pltpu.VMEM_SHARED((_FPAD,), jnp.int32),
            pltpu.VMEM_SHARED((_FPAD,), jnp.int32),
            pltpu.VMEM_SHARED((256,), jnp.int32),
        ],
        compiler_params=pltpu.CompilerParams(needs_layout_passes=False),
    )
    return f(src3, dst3)


_BL = 3200


def _layer_dense_body(agg_ref, h_ref, w1_ref, b1_ref, w2_ref, b2_ref, out_ref):
    t = agg_ref[...].astype(jnp.float32) + h_ref[...]
    u = jnp.maximum(jnp.dot(t, w1_ref[...], preferred_element_type=jnp.float32)
                    + b1_ref[...], 0.0)
    u = jnp.dot(u, w2_ref[...], preferred_element_type=jnp.float32) + b2_ref[...]
    out_ref[...] = h_ref[...] + u


def _layer_dense(agg, h, W1, b1, W2, b2):
    grid = (F // _BL,)
    return pl.pallas_call(
        _layer_dense_body,
        grid=grid,
        in_specs=[
            pl.BlockSpec((_BL, H), lambda i: (i, 0)),
            pl.BlockSpec((_BL, H), lambda i: (i, 0)),
            pl.BlockSpec((H, H), lambda i: (0, 0)),
            pl.BlockSpec((1, H), lambda i: (0, 0)),
            pl.BlockSpec((H, H), lambda i: (0, 0)),
            pl.BlockSpec((1, H), lambda i: (0, 0)),
        ],
        out_specs=pl.BlockSpec((_BL, H), lambda i: (i, 0)),
        out_shape=jax.ShapeDtypeStruct((F, H), jnp.float32),
    )(agg, h, W1, b1.reshape(1, H), W2, b2.reshape(1, H))


_BN = 1000


def _mha_body(hf_ref, lp_ref, wq_ref, wk_ref, wv_ref, wo_ref, bq_ref, bk_ref,
              bv_ref, bo_ref, alpha_ref, ne_ref, bsum_ref, bsq_ref):
    hf = hf_ref[...].reshape(_BN * M, K, H)
    h2 = jnp.sum(hf, axis=1).reshape(_BN, M, H)
    hs = [h2[:, m, :] for m in range(M)]
    wq, wk, wv, wo = wq_ref[...], wk_ref[...], wv_ref[...], wo_ref[...]
    q = [jnp.dot(hs[m], wq, preferred_element_type=jnp.float32) + bq_ref[...]
         for m in range(M)]
    k = [jnp.dot(hs[m], wk, preferred_element_type=jnp.float32) + bk_ref[...]
         for m in range(M)]
    v = [jnp.dot(hs[m], wv, preferred_element_type=jnp.float32) + bv_ref[...]
         for m in range(M)]
    r = lax.broadcasted_iota(jnp.int32, (H, H), 0) // DH
    c = lax.broadcasted_iota(jnp.int32, (H, H), 1) // DH
    headmask = (r == c).astype(jnp.float32)
    scale = 1.0 / math.sqrt(DH)
    alpha = alpha_ref[0, 0]
    bias = [(-alpha) * lp_ref[:, ki:ki + 1] for ki in range(M)]
    ne = jnp.zeros((_BN, H), jnp.float32)
    for qi in range(M):
        sc = [jnp.dot(q[qi] * k[ki], headmask,
                      preferred_element_type=jnp.float32) * scale + bias[ki]
              for ki in range(M)]
        mx = jnp.maximum(jnp.maximum(sc[0], sc[1]), jnp.maximum(sc[2], sc[3]))
        ex = [jnp.exp(s - mx) for s in sc]
        den = ex[0] + ex[1] + ex[2] + ex[3]
        ctx = sum((ex[ki] / den) * v[ki] for ki in range(M))
        ha = jnp.dot(ctx, wo, preferred_element_type=jnp.float32) + bo_ref[...] \
            + hs[qi]
        ne = ne + ha
    ne = ne * (1.0 / M)
    ne_ref[...] = ne

    @pl.when(pl.program_id(0) == 0)
    def _():
        bsum_ref[...] = jnp.zeros_like(bsum_ref)
        bsq_ref[...] = jnp.zeros_like(bsq_ref)

    bsum_ref[...] += jnp.broadcast_to(jnp.sum(ne, axis=0, keepdims=True), (8, H))
    bsq_ref[...] += jnp.broadcast_to(jnp.sum(ne * ne, axis=0, keepdims=True),
                                     (8, H))


def _mha(h, lp2, Wq, Wk, Wv, Wo, bq, bk, bv, bo, alpha):
    grid = (N_TOTAL // _BN,)
    return pl.pallas_call(
        _mha_body,
        grid=grid,
        in_specs=[
            pl.BlockSpec((_BN * M * K, H), lambda i: (i, 0)),
            pl.BlockSpec((_BN, M), lambda i: (i, 0)),
            pl.BlockSpec((H, H), lambda i: (0, 0)),
            pl.BlockSpec((H, H), lambda i: (0, 0)),
            pl.BlockSpec((H, H), lambda i: (0, 0)),
            pl.BlockSpec((H, H), lambda i: (0, 0)),
            pl.BlockSpec((1, H), lambda i: (0, 0)),
            pl.BlockSpec((1, H), lambda i: (0, 0)),
            pl.BlockSpec((1, H), lambda i: (0, 0)),
            pl.BlockSpec((1, H), lambda i: (0, 0)),
            pl.BlockSpec((1, 1), lambda i: (0, 0)),
        ],
        out_specs=[
            pl.BlockSpec((_BN, H), lambda i: (i, 0)),
            pl.BlockSpec((8, H), lambda i: (0, 0)),
            pl.BlockSpec((8, H), lambda i: (0, 0)),
        ],
        out_shape=[
            jax.ShapeDtypeStruct((N_TOTAL, H), jnp.float32),
            jax.ShapeDtypeStruct((8, H), jnp.float32),
            jax.ShapeDtypeStruct((8, H), jnp.float32),
        ],
    )(h, lp2, Wq, Wk, Wv, Wo, bq.reshape(1, H), bk.reshape(1, H),
      bv.reshape(1, H), bo.reshape(1, H), alpha.reshape(1, 1))


def _bnpool_body(ne_ref, mu_ref, inv_ref, g_ref, b_ref, batch_ref, out_ref):
    i = pl.program_id(0)

    @pl.when(i == 0)
    def _():
        out_ref[...] = jnp.zeros_like(out_ref)

    normed = (ne_ref[...] - mu_ref[...]) * inv_ref[...] * g_ref[...] + b_ref[...]
    onehot = (batch_ref[...] == lax.broadcasted_iota(
        jnp.int32, (_BN, NUM_GRAPHS), 1)).astype(jnp.float32)
    out_ref[...] += lax.dot_general(
        onehot, normed, (((0,), (0,)), ((), ())),
        preferred_element_type=jnp.float32)


def _bnpool(ne, mu, inv, gamma, beta, batch2d):
    grid = (N_TOTAL // _BN,)
    return pl.pallas_call(
        _bnpool_body,
        grid=grid,
        in_specs=[
            pl.BlockSpec((_BN, H), lambda i: (i, 0)),
            pl.BlockSpec((1, H), lambda i: (0, 0)),
            pl.BlockSpec((1, H), lambda i: (0, 0)),
            pl.BlockSpec((1, H), lambda i: (0, 0)),
            pl.BlockSpec((1, H), lambda i: (0, 0)),
            pl.BlockSpec((_BN, 1), lambda i: (i, 0)),
        ],
        out_specs=pl.BlockSpec((NUM_GRAPHS, H), lambda i: (0, 0)),
        out_shape=jax.ShapeDtypeStruct((NUM_GRAPHS, H), jnp.float32),
    )(ne, mu, inv, gamma.reshape(1, H), beta.reshape(1, H), batch2d)


def kernel(x, edge_index, edge_attr, nodes_sampled, log_probs, batch,
           atom_table, bond_table, dist_table, logp_w, logp_b, layer_params,
           Wq, Wk, Wv, bq, bk, bv, Wo, bo, ht_alpha, bn_gamma, bn_beta):
    src, dst = edge_index[0], edge_index[1]
    node_ids = nodes_sampled.reshape(-1)
    lp = log_probs

    comb = (atom_table[:, None, :] + dist_table[None, :, :]).reshape(-1, H)

    dist = _bfs_dist(src, dst)

    cidx = x[node_ids] * (MAX_DIST + 1) + dist
    base = jnp.take(comb, cidx, axis=0)
    logp_pe = jax.nn.relu(lp[:, None] * logp_w + logp_b)
    h = (base.reshape(S, K, H) + logp_pe[:, None, :]).reshape(F, H)

    ea = jnp.take(bond_table, jnp.clip(edge_attr - 1, 0, EDGE_DIM - 1),
                  axis=0).astype(jnp.bfloat16)
    for (W1, b1, W2, b2) in layer_params:
        hb = h.astype(jnp.bfloat16)
        msg = jax.nn.relu(jnp.take(hb, src, axis=0) + ea)
        agg = jnp.zeros((F, H), jnp.bfloat16).at[dst].add(msg)
        h = _layer_dense(agg, h, W1, b1, W2, b2)

    lp2 = lp.reshape(N_TOTAL, M)
    ne, bsum, bsq = _mha(h, lp2, Wq, Wk, Wv, Wo, bq, bk, bv, bo, ht_alpha)
    mu = bsum[0:1, :] / N_TOTAL
    var = bsq[0:1, :] / N_TOTAL - mu * mu
    inv = 1.0 / jnp.sqrt(var + 1e-5)
    return _bnpool(ne, mu, inv, bn_gamma, bn_beta, batch.reshape(N_TOTAL, 1))


# speedup vs baseline: 1.1044x; 1.1044x over previous
import functools
import math

import jax
import jax.numpy as jnp
from jax import lax
from jax.experimental import pallas as pl
from jax.experimental.pallas import tpu as pltpu
from jax.experimental.pallas import tpu_sc as plsc

N_TOTAL = 10000
M = 4
K = 8
S = N_TOTAL * M
F = S * K
E = 500000
H = 128
NH = 4
DH = H // NH
MAX_DIST = 32
EDGE_DIM = 8
NUM_GRAPHS = 64

_EPAD = 524288
_EPT = _EPAD // 16
_FPAD = 327680
_NPT = _FPAD // 16
_WORDS = _FPAD // 32
_WPT = _WORDS // 16
_CH = 8192


def _bfs_body(src_hbm, dst_hbm, out_hbm, src_c, dst_c, c_v, dist_v,
              racc_v, cnt_v, cntall_v, racc_sp, rprev_sp, cnt_sp, *,
              rounds=MAX_DIST - 1):
    cid = lax.axis_index("c")
    sid = lax.axis_index("s")
    lane = lax.iota(jnp.int32, 16)
    rootpat = jnp.where(lane % 8 == 0, 1, 0).astype(jnp.int32)
    unreach = jnp.where(lane % 8 == 0, 0, 1).astype(jnp.int32)

    def fill(i, _):
        racc_v[pl.ds(i * 16, 16)] = rootpat
        dist_v[pl.ds(i * 16, 16)] = unreach
        return 0

    lax.fori_loop(0, _NPT // 16, fill, 0)
    pltpu.sync_copy(racc_v, racc_sp.at[pl.ds(sid * _NPT, _NPT)])
    pltpu.sync_copy(racc_v, rprev_sp.at[pl.ds(sid * _NPT, _NPT)])
    plsc.subcore_barrier()

    def round_body(carry):
        t, prev_total, _ = carry
        for j in range(_EPT // _CH):
            pltpu.sync_copy(src_hbm.at[sid, j], src_c)
            pltpu.sync_copy(dst_hbm.at[sid, j], dst_c)
            pltpu.sync_copy(rprev_sp.at[src_c], c_v)
            pltpu.sync_copy(c_v, racc_sp.at[dst_c], add=True)
        plsc.subcore_barrier()
        pltpu.sync_copy(racc_sp.at[pl.ds(sid * _NPT, _NPT)], racc_v)

        def dstep(i, acc):
            b = jnp.minimum(racc_v[pl.ds(i * 16, 16)], 1)
            dist_v[pl.ds(i * 16, 16)] = dist_v[pl.ds(i * 16, 16)] + 1 - b
            racc_v[pl.ds(i * 16, 16)] = b
            return acc + b

        acc = lax.fori_loop(0, _NPT // 16, dstep, jnp.zeros((16,), jnp.int32))
        pltpu.sync_copy(racc_v, rprev_sp.at[pl.ds(sid * _NPT, _NPT)])
        cnt_v[pl.ds(0, 16)] = jnp.broadcast_to(jnp.sum(acc, axis=0), (16,))
        pltpu.sync_copy(cnt_v, cnt_sp.at[pl.ds(sid * 16, 16)])
        plsc.subcore_barrier()
        pltpu.sync_copy(cnt_sp, cntall_v)
        total = jnp.zeros((16,), jnp.int32)
        for i in range(16):
            total = total + cntall_v[pl.ds(i * 16, 16)]
        total = jnp.sum(total, axis=0)
        plsc.subcore_barrier()
        return t + 1, total, prev_total != total

    def round_cond(carry):
        t, _, changed = carry
        return jnp.logical_and(t < rounds, changed)

    t_done, _, _ = lax.while_loop(round_cond, round_body,
                                  (jnp.int32(0), jnp.int32(-1), True))
    rem = rounds - t_done

    def fixup(i, _):
        b = racc_v[pl.ds(i * 16, 16)]
        dist_v[pl.ds(i * 16, 16)] = dist_v[pl.ds(i * 16, 16)] + rem * (1 - b)
        return 0

    lax.fori_loop(0, _NPT // 16, fixup, 0)

    @pl.when(cid == 0)
    def _():
        @pl.when(sid < 15)
        def _():
            pltpu.sync_copy(dist_v, out_hbm.at[pl.ds(sid * _NPT, _NPT)])

        @pl.when(sid == 15)
        def _():
            pltpu.sync_copy(dist_v.at[pl.ds(0, F - 15 * _NPT)],
                            out_hbm.at[pl.ds(15 * _NPT, F - 15 * _NPT)])


def _bfs_dist(src, dst, rounds=MAX_DIST - 1):
    pad = _EPAD - E
    srcp = jnp.concatenate([src, jnp.zeros((pad,), src.dtype)])
    dstp = jnp.concatenate([dst, jnp.zeros((pad,), dst.dtype)])
    src3 = srcp.reshape(16, _EPT // _CH, _CH).astype(jnp.int32)
    dst3 = dstp.reshape(16, _EPT // _CH, _CH).astype(jnp.int32)
    mesh = plsc.VectorSubcoreMesh(core_axis_name="c", subcore_axis_name="s")
    f = pl.kernel(
        functools.partial(_bfs_body, rounds=rounds),
        out_type=jax.ShapeDtypeStruct((F,), jnp.int32),
        mesh=mesh,
        scratch_types=[
            pltpu.VMEM((_CH,), jnp.int32),
            pltpu.VMEM((_CH,), jnp.int32),
            pltpu.VMEM((_CH,), jnp.int32),
            pltpu.VMEM((_NPT,), jnp.int32),
            pltpu.VMEM((_NPT,), jnp.int32),
            pltpu.VMEM((16,), jnp.int32),
            pltpu.VMEM((256,), jnp.int32),
            pltpu.VMEM_SHARED((_FPAD,), jnp.int32),
            pltpu.VMEM_SHARED((_FPAD,), jnp.int32),
            pltpu.VMEM_SHARED((256,), jnp.int32),
        ],
        compiler_params=pltpu.CompilerParams(needs_layout_passes=False),
    )
    return f(src3, dst3)


_BL = 3200


def _layer_dense_body(agg_ref, h_ref, w1_ref, b1_ref, w2_ref, b2_ref, out_ref):
    t = agg_ref[...] + h_ref[...]
    u = jnp.maximum(jnp.dot(t, w1_ref[...], preferred_element_type=jnp.float32)
                    + b1_ref[...], 0.0)
    u = jnp.dot(u, w2_ref[...], preferred_element_type=jnp.float32) + b2_ref[...]
    out_ref[...] = h_ref[...] + u


def _layer_dense(agg, h, W1, b1, W2, b2):
    grid = (F // _BL,)
    return pl.pallas_call(
        _layer_dense_body,
        grid=grid,
        in_specs=[
            pl.BlockSpec((_BL, H), lambda i: (i, 0)),
            pl.BlockSpec((_BL, H), lambda i: (i, 0)),
            pl.BlockSpec((H, H), lambda i: (0, 0)),
            pl.BlockSpec((1, H), lambda i: (0, 0)),
            pl.BlockSpec((H, H), lambda i: (0, 0)),
            pl.BlockSpec((1, H), lambda i: (0, 0)),
        ],
        out_specs=pl.BlockSpec((_BL, H), lambda i: (i, 0)),
        out_shape=jax.ShapeDtypeStruct((F, H), jnp.float32),
    )(agg, h, W1, b1.reshape(1, H), W2, b2.reshape(1, H))


_BN = 1000


def _mha_body(hf_ref, lp_ref, wq_ref, wk_ref, wv_ref, wo_ref, bq_ref, bk_ref,
              bv_ref, bo_ref, alpha_ref, ne_ref, bsum_ref, bsq_ref):
    hf = hf_ref[...].reshape(_BN * M, K, H)
    h2 = jnp.sum(hf, axis=1).reshape(_BN, M, H)
    hs = [h2[:, m, :] for m in range(M)]
    wq, wk, wv, wo = wq_ref[...], wk_ref[...], wv_ref[...], wo_ref[...]
    q = [jnp.dot(hs[m], wq, preferred_element_type=jnp.float32) + bq_ref[...]
         for m in range(M)]
    k = [jnp.dot(hs[m], wk, preferred_element_type=jnp.float32) + bk_ref[...]
         for m in range(M)]
    v = [jnp.dot(hs[m], wv, preferred_element_type=jnp.float32) + bv_ref[...]
         for m in range(M)]
    r = lax.broadcasted_iota(jnp.int32, (H, H), 0) // DH
    c = lax.broadcasted_iota(jnp.int32, (H, H), 1) // DH
    headmask = (r == c).astype(jnp.float32)
    scale = 1.0 / math.sqrt(DH)
    alpha = alpha_ref[0, 0]
    bias = [(-alpha) * lp_ref[:, ki:ki + 1] for ki in range(M)]
    ne = jnp.zeros((_BN, H), jnp.float32)
    for qi in range(M):
        sc = [jnp.dot(q[qi] * k[ki], headmask,
                      preferred_element_type=jnp.float32) * scale + bias[ki]
              for ki in range(M)]
        mx = jnp.maximum(jnp.maximum(sc[0], sc[1]), jnp.maximum(sc[2], sc[3]))
        ex = [jnp.exp(s - mx) for s in sc]
        den = ex[0] + ex[1] + ex[2] + ex[3]
        ctx = sum((ex[ki] / den) * v[ki] for ki in range(M))
        ha = jnp.dot(ctx, wo, preferred_element_type=jnp.float32) + bo_ref[...] \
            + hs[qi]
        ne = ne + ha
    ne = ne * (1.0 / M)
    ne_ref[...] = ne

    @pl.when(pl.program_id(0) == 0)
    def _():
        bsum_ref[...] = jnp.zeros_like(bsum_ref)
        bsq_ref[...] = jnp.zeros_like(bsq_ref)

    bsum_ref[...] += jnp.broadcast_to(jnp.sum(ne, axis=0, keepdims=True), (8, H))
    bsq_ref[...] += jnp.broadcast_to(jnp.sum(ne * ne, axis=0, keepdims=True),
                                     (8, H))


def _mha(h, lp2, Wq, Wk, Wv, Wo, bq, bk, bv, bo, alpha):
    grid = (N_TOTAL // _BN,)
    return pl.pallas_call(
        _mha_body,
        grid=grid,
        in_specs=[
            pl.BlockSpec((_BN * M * K, H), lambda i: (i, 0)),
            pl.BlockSpec((_BN, M), lambda i: (i, 0)),
            pl.BlockSpec((H, H), lambda i: (0, 0)),
            pl.BlockSpec((H, H), lambda i: (0, 0)),
            pl.BlockSpec((H, H), lambda i: (0, 0)),
            pl.BlockSpec((H, H), lambda i: (0, 0)),
            pl.BlockSpec((1, H), lambda i: (0, 0)),
            pl.BlockSpec((1, H), lambda i: (0, 0)),
            pl.BlockSpec((1, H), lambda i: (0, 0)),
            pl.BlockSpec((1, H), lambda i: (0, 0)),
            pl.BlockSpec((1, 1), lambda i: (0, 0)),
        ],
        out_specs=[
            pl.BlockSpec((_BN, H), lambda i: (i, 0)),
            pl.BlockSpec((8, H), lambda i: (0, 0)),
            pl.BlockSpec((8, H), lambda i: (0, 0)),
        ],
        out_shape=[
            jax.ShapeDtypeStruct((N_TOTAL, H), jnp.float32),
            jax.ShapeDtypeStruct((8, H), jnp.float32),
            jax.ShapeDtypeStruct((8, H), jnp.float32),
        ],
    )(h, lp2, Wq, Wk, Wv, Wo, bq.reshape(1, H), bk.reshape(1, H),
      bv.reshape(1, H), bo.reshape(1, H), alpha.reshape(1, 1))


def _bnpool_body(ne_ref, mu_ref, inv_ref, g_ref, b_ref, batch_ref, out_ref):
    i = pl.program_id(0)

    @pl.when(i == 0)
    def _():
        out_ref[...] = jnp.zeros_like(out_ref)

    normed = (ne_ref[...] - mu_ref[...]) * inv_ref[...] * g_ref[...] + b_ref[...]
    onehot = (batch_ref[...] == lax.broadcasted_iota(
        jnp.int32, (_BN, NUM_GRAPHS), 1)).astype(jnp.float32)
    out_ref[...] += lax.dot_general(
        onehot, normed, (((0,), (0,)), ((), ())),
        preferred_element_type=jnp.float32)


def _bnpool(ne, mu, inv, gamma, beta, batch2d):
    grid = (N_TOTAL // _BN,)
    return pl.pallas_call(
        _bnpool_body,
        grid=grid,
        in_specs=[
            pl.BlockSpec((_BN, H), lambda i: (i, 0)),
            pl.BlockSpec((1, H), lambda i: (0, 0)),
            pl.BlockSpec((1, H), lambda i: (0, 0)),
            pl.BlockSpec((1, H), lambda i: (0, 0)),
            pl.BlockSpec((1, H), lambda i: (0, 0)),
            pl.BlockSpec((_BN, 1), lambda i: (i, 0)),
        ],
        out_specs=pl.BlockSpec((NUM_GRAPHS, H), lambda i: (0, 0)),
        out_shape=jax.ShapeDtypeStruct((NUM_GRAPHS, H), jnp.float32),
    )(ne, mu, inv, gamma.reshape(1, H), beta.reshape(1, H), batch2d)


def kernel(x, edge_index, edge_attr, nodes_sampled, log_probs, batch,
           atom_table, bond_table, dist_table, logp_w, logp_b, layer_params,
           Wq, Wk, Wv, bq, bk, bv, Wo, bo, ht_alpha, bn_gamma, bn_beta):
    src, dst = edge_index[0], edge_index[1]
    node_ids = nodes_sampled.reshape(-1)
    lp = log_probs

    comb = (atom_table[:, None, :] + dist_table[None, :, :]).reshape(-1, H)

    dist = _bfs_dist(src, dst)

    cidx = x[node_ids] * (MAX_DIST + 1) + dist
    base = jnp.take(comb, cidx, axis=0)
    logp_pe = jax.nn.relu(lp[:, None] * logp_w + logp_b)
    h = (base.reshape(S, K, H) + logp_pe[:, None, :]).reshape(F, H)

    ea = jnp.take(bond_table, jnp.clip(edge_attr - 1, 0, EDGE_DIM - 1), axis=0)
    for (W1, b1, W2, b2) in layer_params:
        msg = jax.nn.relu(jnp.take(h, src, axis=0) + ea)
        agg = jnp.zeros_like(h).at[dst].add(msg)
        h = _layer_dense(agg, h, W1, b1, W2, b2)

    lp2 = lp.reshape(N_TOTAL, M)
    ne, bsum, bsq = _mha(h, lp2, Wq, Wk, Wv, Wo, bq, bk, bv, bo, ht_alpha)
    mu = bsum[0:1, :] / N_TOTAL
    var = bsq[0:1, :] / N_TOTAL - mu * mu
    inv = 1.0 / jnp.sqrt(var + 1e-5)
    return _bnpool(ne, mu, inv, bn_gamma, bn_beta, batch.reshape(N_TOTAL, 1))
